# baseline (speedup 1.0000x reference)
import functools

import jax
import jax.numpy as jnp
from jax import lax
from jax.experimental import pallas as pl
from jax.experimental.pallas import tpu as pltpu
from jax.experimental.pallas import tpu_sc as plsc

VOCAB = 50257
EMBED = 256
B, S = 1024, 200
NTOK = B * S
NC, NS, L = 2, 16, 16
NW = NC * NS
TPW = NTOK // NW
CHUNK = 32
NCH = TPW // CHUNK
NBUF = 4
NJ = EMBED // L


def _allsum16(v):
    lanes = lax.iota(jnp.int32, L)
    for sh in (8, 4, 2, 1):
        perm = jnp.bitwise_xor(lanes, sh)
        v = v + v.at[perm].get(mode="promise_in_bounds")
    return v


def _rsqrt16(x):
    i = lax.bitcast_convert_type(x, jnp.int32)
    y = lax.bitcast_convert_type(jnp.int32(0x5F3759DF) - (i >> 1), jnp.float32)
    for _ in range(3):
        y = y * (1.5 - 0.5 * x * y * y)
    return y


def _body(ids_hbm, table_hbm, pospm_hbm, gamma_hbm, beta_hbm, out_hbm,
          idx_v, pos_v, g_v, b_v, mb_v, ib_v,
          tok0, tok1, tok2, tok3,
          gsem0, gsem1, gsem2, gsem3, ssem0, ssem1, ssem2, ssem3):
    wid = lax.axis_index("s") * NC + lax.axis_index("c")
    base = wid * TPW
    bufs = [tok0, tok1, tok2, tok3]
    gsems = [gsem0, gsem1, gsem2, gsem3]
    ssems = [ssem0, ssem1, ssem2, ssem3]

    pltpu.sync_copy(ids_hbm.at[wid], idx_v)
    pltpu.sync_copy(pospm_hbm, pos_v)
    pltpu.sync_copy(gamma_hbm, g_v)
    pltpu.sync_copy(beta_hbm, b_v)

    def gcopy(g, k):
        return pltpu.make_async_copy(
            table_hbm.at[idx_v.at[g]], bufs[k], gsems[k])

    def scopy(g, k):
        return pltpu.make_async_copy(
            bufs[k], out_hbm.at[pl.ds(base + g * CHUNK, CHUNK)], ssems[k])

    def compute(buf, g):
        @plsc.parallel_loop(0, CHUNK, unroll=2)
        def stat_row(r):
            rl = r % L
            idv = idx_v[g, pl.ds((r // L) * L, L)]
            mi = idv.at[jnp.full((L,), rl, jnp.int32)].get(
                mode="promise_in_bounds")
            m = jnp.where(mi != 0, jnp.float32(1.0), jnp.float32(0.0))
            p = (g * CHUNK + r) % S
            a = [jnp.zeros((L,), jnp.float32) for _ in range(4)]
            q = [jnp.zeros((L,), jnp.float32) for _ in range(4)]
            for j in range(NJ):
                t = buf[r, pl.ds(j * L, L)] * m + pos_v[p, pl.ds(j * L, L)]
                buf[r, pl.ds(j * L, L)] = t
                a[j % 4] = a[j % 4] + t
                q[j % 4] = q[j % 4] + t * t
            acc = (a[0] + a[1]) + (a[2] + a[3])
            acc2 = (q[0] + q[1]) + (q[2] + q[3])
            meanv = _allsum16(acc) * (1.0 / EMBED)
            varv = _allsum16(acc2) * (1.0 / EMBED) - meanv * meanv
            mb_v[r] = meanv
            ib_v[r] = _rsqrt16(varv + 1e-5)

        gs = [g_v[pl.ds(j * L, L)] for j in range(NJ)]
        bs = [b_v[pl.ds(j * L, L)] for j in range(NJ)]

        @plsc.parallel_loop(0, CHUNK, unroll=2)
        def norm_row(r):
            meanv = mb_v[r]
            inv = ib_v[r]
            for j in range(NJ):
                t = buf[r, pl.ds(j * L, L)]
                buf[r, pl.ds(j * L, L)] = (t - meanv) * inv * gs[j] + bs[j]

    gcopy(0, 0).start()
    gcopy(1, 1).start()

    def ring(i, c):
        for k in range(NBUF):
            g = NBUF * i + k

            @pl.when(g >= 2)
            def _():
                scopy(g - 2, (k - 2) % NBUF).wait()

            @pl.when(g + 2 < NCH)
            def _():
                gcopy(g + 2, (k + 2) % NBUF).start()

            gcopy(g, k).wait()
            compute(bufs[k], g)
            scopy(g, k).start()
        return c

    lax.fori_loop(0, NCH // NBUF, ring, 0)
    scopy(NCH - 2, (NCH - 2) % NBUF).wait()
    scopy(NCH - 1, (NCH - 1) % NBUF).wait()


@functools.partial(jax.jit, static_argnames=())
def _launch(ids, table, pospm, gamma, beta):
    mesh = plsc.VectorSubcoreMesh(core_axis_name="c", subcore_axis_name="s")
    return pl.kernel(
        _body,
        out_type=jax.ShapeDtypeStruct((NTOK, EMBED), jnp.float32),
        mesh=mesh,
        scratch_types=[
            pltpu.VMEM((NCH, CHUNK), jnp.int32),
            pltpu.VMEM((S, EMBED), jnp.float32),
            pltpu.VMEM((EMBED,), jnp.float32),
            pltpu.VMEM((EMBED,), jnp.float32),
            pltpu.VMEM((CHUNK, L), jnp.float32),
            pltpu.VMEM((CHUNK, L), jnp.float32),
            pltpu.VMEM((CHUNK, EMBED), jnp.float32),
            pltpu.VMEM((CHUNK, EMBED), jnp.float32),
            pltpu.VMEM((CHUNK, EMBED), jnp.float32),
            pltpu.VMEM((CHUNK, EMBED), jnp.float32),
            pltpu.SemaphoreType.DMA,
            pltpu.SemaphoreType.DMA,
            pltpu.SemaphoreType.DMA,
            pltpu.SemaphoreType.DMA,
            pltpu.SemaphoreType.DMA,
            pltpu.SemaphoreType.DMA,
            pltpu.SemaphoreType.DMA,
            pltpu.SemaphoreType.DMA,
        ],
    )(ids, table, pospm, gamma, beta)


def kernel(input_ids, token_table, pos_table, modality, gamma, beta):
    ids = input_ids.astype(jnp.int32).reshape(NW, NCH, CHUNK)
    pospm = (pos_table[:S] + modality.reshape(1, EMBED)).astype(jnp.float32)
    out = _launch(ids, token_table, pospm, gamma, beta)
    embeddings = out.reshape(B, S, EMBED)
    attention_mask = jnp.ones_like(input_ids)
    return (embeddings, attention_mask)


# speedup vs baseline: 1.5073x; 1.5073x over previous
import functools

import jax
import jax.numpy as jnp
from jax import lax
from jax.experimental import pallas as pl
from jax.experimental.pallas import tpu as pltpu
from jax.experimental.pallas import tpu_sc as plsc

VOCAB = 50257
EMBED = 256
B, S = 1024, 200
NTOK = B * S
NC, NS, L = 2, 16, 16
NW = NC * NS
TPW = NTOK // NW
CHUNK = 80
NCH = TPW // CHUNK
NJ = EMBED // L


def _allsum16(v):
    lanes = lax.iota(jnp.int32, L)
    for sh in (8, 4, 2, 1):
        perm = jnp.bitwise_xor(lanes, sh)
        v = v + v.at[perm].get(mode="promise_in_bounds")
    return v


def _rsqrt16(x):
    i = lax.bitcast_convert_type(x, jnp.int32)
    y = lax.bitcast_convert_type(jnp.int32(0x5F3759DF) - (i >> 1), jnp.float32)
    for _ in range(3):
        y = y * (1.5 - 0.5 * x * y * y)
    return y


def _body(ids_hbm, table_hbm, pospm_hbm, gamma_hbm, beta_hbm, out_hbm,
          idx_v, pos_v, g_v, b_v, mb_v, ib_v, tok0, tok1,
          gsem0, gsem1, ssem0, ssem1):
    wid = lax.axis_index("s") * NC + lax.axis_index("c")
    base = wid * TPW

    pltpu.sync_copy(ids_hbm.at[wid], idx_v)
    pltpu.sync_copy(pospm_hbm, pos_v)
    pltpu.sync_copy(gamma_hbm, g_v)
    pltpu.sync_copy(beta_hbm, b_v)

    def gcopy(g, buf, sem):
        return pltpu.make_async_copy(table_hbm.at[idx_v.at[g]], buf, sem)

    def scopy(g, buf, sem):
        return pltpu.make_async_copy(
            buf, out_hbm.at[pl.ds(base + g * CHUNK, CHUNK)], sem)

    def stat_pass(buf, g):
        @plsc.parallel_loop(0, CHUNK, unroll=2)
        def stat_row(r):
            rl = r % L
            idv = idx_v[g, pl.ds((r // L) * L, L)]
            mi = idv.at[jnp.full((L,), rl, jnp.int32)].get(
                mode="promise_in_bounds")
            m = jnp.where(mi != 0, jnp.float32(1.0), jnp.float32(0.0))
            p = (g * CHUNK + r) % S
            a = [jnp.zeros((L,), jnp.float32) for _ in range(4)]
            q = [jnp.zeros((L,), jnp.float32) for _ in range(4)]
            for j in range(NJ):
                t = buf[r, pl.ds(j * L, L)] * m + pos_v[p, pl.ds(j * L, L)]
                buf[r, pl.ds(j * L, L)] = t
                a[j % 4] = a[j % 4] + t
                q[j % 4] = q[j % 4] + t * t
            acc = (a[0] + a[1]) + (a[2] + a[3])
            acc2 = (q[0] + q[1]) + (q[2] + q[3])
            meanv = _allsum16(acc) * (1.0 / EMBED)
            varv = _allsum16(acc2) * (1.0 / EMBED) - meanv * meanv
            mb_v[r] = meanv
            ib_v[r] = _rsqrt16(varv + 1e-5)

    def norm_pass(buf, g):
        gs = [g_v[pl.ds(j * L, L)] for j in range(NJ)]
        bs = [b_v[pl.ds(j * L, L)] for j in range(NJ)]

        @plsc.parallel_loop(0, CHUNK, unroll=2)
        def norm_row(r):
            meanv = mb_v[r]
            inv = ib_v[r]
            for j in range(NJ):
                t = buf[r, pl.ds(j * L, L)]
                buf[r, pl.ds(j * L, L)] = (t - meanv) * inv * gs[j] + bs[j]

    gcopy(0, tok0, gsem0).start()

    def pair(i, c):
        g0 = 2 * i
        g1 = g0 + 1

        gcopy(g0, tok0, gsem0).wait()
        stat_pass(tok0, g0)

        @pl.when(i > 0)
        def _():
            scopy(g0 - 1, tok1, ssem1).wait()
        gcopy(g1, tok1, gsem1).start()
        norm_pass(tok0, g0)
        scopy(g0, tok0, ssem0).start()

        gcopy(g1, tok1, gsem1).wait()
        stat_pass(tok1, g1)

        scopy(g0, tok0, ssem0).wait()

        @pl.when(i < NCH // 2 - 1)
        def _():
            gcopy(g0 + 2, tok0, gsem0).start()
        norm_pass(tok1, g1)
        scopy(g1, tok1, ssem1).start()
        return c

    lax.fori_loop(0, NCH // 2, pair, 0)
    scopy(NCH - 1, tok1, ssem1).wait()


@functools.partial(jax.jit, static_argnames=())
def _launch(ids, table, pospm, gamma, beta):
    mesh = plsc.VectorSubcoreMesh(core_axis_name="c", subcore_axis_name="s")
    return pl.kernel(
        _body,
        out_type=jax.ShapeDtypeStruct((NTOK, EMBED), jnp.float32),
        mesh=mesh,
        scratch_types=[
            pltpu.VMEM((NCH, CHUNK), jnp.int32),
            pltpu.VMEM((S, EMBED), jnp.float32),
            pltpu.VMEM((EMBED,), jnp.float32),
            pltpu.VMEM((EMBED,), jnp.float32),
            pltpu.VMEM((CHUNK, L), jnp.float32),
            pltpu.VMEM((CHUNK, L), jnp.float32),
            pltpu.VMEM((CHUNK, EMBED), jnp.float32),
            pltpu.VMEM((CHUNK, EMBED), jnp.float32),
            pltpu.SemaphoreType.DMA,
            pltpu.SemaphoreType.DMA,
            pltpu.SemaphoreType.DMA,
            pltpu.SemaphoreType.DMA,
        ],
    )(ids, table, pospm, gamma, beta)


def kernel(input_ids, token_table, pos_table, modality, gamma, beta):
    ids = input_ids.astype(jnp.int32).reshape(NW, NCH, CHUNK)
    pospm = (pos_table[:S] + modality.reshape(1, EMBED)).astype(jnp.float32)
    out = _launch(ids, token_table, pospm, gamma, beta)
    embeddings = out.reshape(B, S, EMBED)
    attention_mask = jnp.ones_like(input_ids)
    return (embeddings, attention_mask)
